# baseline (speedup 1.0000x reference)
import functools

import jax
import jax.numpy as jnp
from jax import lax
from jax.experimental import pallas as pl
from jax.experimental.pallas import tpu as pltpu
from jax.experimental.pallas import tpu_sc as plsc

_N = 1_000_000
_R, _C = 64, 15625

_NC, _NS = 2, 16
_NW = _NC * _NS
_CHUNK = 31248
_WIN = 31312
_NVEC = 1953
_TAIL_VECS = 4

_gumbel_cache = []


def _gumbel():
    if not _gumbel_cache:
        g = jax.random.gumbel(jax.random.key(42), (1, _N), jnp.float32)
        _gumbel_cache.append(jnp.reshape(g, (_N,)))
    return _gumbel_cache[0]


def _sc_body(t_hbm, g_hbm, out_hbm, tv, gv, pv):
    c = lax.axis_index("c")
    s = lax.axis_index("s")
    w = s * _NC + c
    base = w * _CHUNK
    pltpu.sync_copy(t_hbm.at[pl.ds(base, _WIN)], tv)
    pltpu.sync_copy(g_hbm.at[pl.ds(base, _WIN)], gv)

    lanes = lax.iota(jnp.int32, 16)
    neg_inf = jnp.full((16,), -jnp.inf, jnp.float32)
    zero = jnp.zeros((16,), jnp.float32)
    big_i = jnp.full((16,), 2**30, jnp.int32)

    def step(i, k, acc):
        ssum, bv, bt, bi = acc
        x = tv[pl.ds((i + k) * 16, 16)]
        g = gv[pl.ds((i + k) * 16, 16)]
        y = x + g
        idx = (base + (i + k) * 16) + lanes
        upd = y > bv
        return (
            ssum + jnp.exp(x),
            jnp.maximum(y, bv),
            jnp.where(upd, x, bt),
            jnp.where(upd, idx, bi),
        )

    init3 = tuple((zero, neg_inf, zero, big_i) for _ in range(3))

    @plsc.parallel_loop(0, _NVEC, step=3, unroll=2, carry=init3)
    def loop(i, accs):
        return tuple(step(i, k, accs[k]) for k in range(3))

    accs = loop

    def merge(a, b):
        sa, va, ta, ia = a
        sb, vb, tb, ib = b
        upd = (vb > va) | ((vb == va) & (ib < ia))
        return (
            sa + sb,
            jnp.maximum(va, vb),
            jnp.where(upd, tb, ta),
            jnp.where(upd, ib, ia),
        )

    acc = merge(merge(accs[0], accs[1]), accs[2])

    @pl.when(w == _NW - 1)
    def _():
        a = acc
        for k in range(_TAIL_VECS):
            a = step(_NVEC, k, a)
        pv[pl.ds(0, 16)] = a[0]
        pv[pl.ds(16, 16)] = a[1]
        pv[pl.ds(32, 16)] = a[2]
        pv[pl.ds(48, 16)] = a[3].astype(jnp.float32)

    @pl.when(w != _NW - 1)
    def _():
        pv[pl.ds(0, 16)] = acc[0]
        pv[pl.ds(16, 16)] = acc[1]
        pv[pl.ds(32, 16)] = acc[2]
        pv[pl.ds(48, 16)] = acc[3].astype(jnp.float32)

    pltpu.sync_copy(pv, out_hbm.at[w])


_sc_partials_cache = []


def _sc_partials(t1, g1):
    if not _sc_partials_cache:
        _sc_partials_cache.append(
            functools.partial(
                pl.kernel,
                out_type=jax.ShapeDtypeStruct((_NW, 64), jnp.float32),
                mesh=plsc.VectorSubcoreMesh(
                    core_axis_name="c", subcore_axis_name="s",
                    num_cores=_NC, num_subcores=_NS,
                ),
                scratch_types=[
                    pltpu.VMEM((_WIN,), jnp.float32),
                    pltpu.VMEM((_WIN,), jnp.float32),
                    pltpu.VMEM((64,), jnp.float32),
                ],
            )(_sc_body)
        )
    return _sc_partials_cache[0](t1, g1)


import numpy as _np

_mm = _np.linspace(1.0, 2.0, 4001)
_LN_COEFS = [float(c) for c in _np.polyfit(_mm, _np.log(_mm), 9)]
_LN2 = float(_np.log(2.0))


def _softln_scalar(s):
    bits = lax.bitcast_convert_type(s, jnp.int32)
    e = ((bits >> 23) - 127).astype(jnp.float32)
    mbits = (bits & jnp.int32(0x7FFFFF)) | jnp.int32(127 << 23)
    mv = lax.bitcast_convert_type(mbits, jnp.float32)
    acc = jnp.float32(_LN_COEFS[0])
    for c in _LN_COEFS[1:]:
        acc = acc * mv + jnp.float32(c)
    return acc + e * jnp.float32(_LN2)


_REDV = 3


def _sc_map_body(t_hbm, p_hbm, out_hbm, tv, ov, pv):
    c = lax.axis_index("c")
    s = lax.axis_index("s")
    w = s * _NC + c
    base = w * _CHUNK
    pltpu.sync_copy(t_hbm.at[pl.ds(base, _WIN)], tv)
    pltpu.sync_copy(p_hbm, pv)

    ssum = jnp.zeros((16,), jnp.float32)
    bv = jnp.full((16,), -jnp.inf, jnp.float32)
    bt = jnp.zeros((16,), jnp.float32)
    bi = jnp.full((16,), 3e38, jnp.float32)
    for j in range(_NW):
        sj = pv[j, pl.ds(0, 16)]
        vj = pv[j, pl.ds(16, 16)]
        tj = pv[j, pl.ds(32, 16)]
        ij = pv[j, pl.ds(48, 16)]
        ssum = ssum + sj
        upd = (vj > bv) | ((vj == bv) & (ij < bi))
        bv = jnp.maximum(vj, bv)
        bt = jnp.where(upd, tj, bt)
        bi = jnp.where(upd, ij, bi)

    s_tot = jnp.float32(0.0)
    for k in range(16):
        s_tot = s_tot + ssum[k]
    m = jnp.float32(-jnp.inf)
    imin = jnp.float32(3e38)
    tval = jnp.float32(0.0)
    for k in range(16):
        v_k = bv[k]
        t_k = bt[k]
        i_k = bi[k]
        upd = (v_k > m) | ((v_k == m) & (i_k < imin))
        m = jnp.where(upd, v_k, m)
        tval = jnp.where(upd, t_k, tval)
        imin = jnp.where(upd, i_k, imin)

    lse = _softln_scalar(s_tot)
    cadd = tval - 2.0 * lse

    def mstep(i, k):
        x = tv[pl.ds((i + k) * 16, 16)]
        ov[pl.ds((i + k) * 16, 16)] = (cadd + x) + jnp.exp(x - lse)

    @plsc.parallel_loop(0, _NVEC, step=3, unroll=2)
    def mloop(i):
        for k in range(3):
            mstep(i, k)

    pltpu.sync_copy(ov.at[pl.ds(0, _CHUNK)], out_hbm.at[pl.ds(base, _CHUNK)])

    @pl.when(w == _NW - 1)
    def _():
        for k in range(_TAIL_VECS):
            mstep(_NVEC, k)
        pltpu.sync_copy(
            ov.at[pl.ds(_CHUNK, _TAIL_VECS * 16)],
            out_hbm.at[pl.ds(base + _CHUNK, _TAIL_VECS * 16)],
        )


_sc_map_cache = []


def _sc_map(t1, partials):
    if not _sc_map_cache:
        _sc_map_cache.append(
            functools.partial(
                pl.kernel,
                out_type=jax.ShapeDtypeStruct((_N,), jnp.float32),
                mesh=plsc.VectorSubcoreMesh(
                    core_axis_name="c", subcore_axis_name="s",
                    num_cores=_NC, num_subcores=_NS,
                ),
                scratch_types=[
                    pltpu.VMEM((_WIN,), jnp.float32),
                    pltpu.VMEM((_WIN,), jnp.float32),
                    pltpu.VMEM((_NW, 64), jnp.float32),
                ],
            )(_sc_map_body)
        )
    return _sc_map_cache[0](t1, partials)


def kernel(t):
    t1 = jnp.reshape(t, (_N,))
    partials = _sc_partials(t1, _gumbel())
    out = _sc_map(t1, partials)
    return jnp.reshape(out, (1, _N))


# speedup vs baseline: 1.2578x; 1.2578x over previous
import jax
import jax.numpy as jnp
from jax.experimental import pallas as pl
from jax.experimental.pallas import tpu as pltpu

_N = 1_000_000
_R, _C = 64, 15625
_BR = 8
_NB = _R // _BR

_gumbel_cache = []


def _gumbel():
    if not _gumbel_cache:
        g = jax.random.gumbel(jax.random.key(42), (1, _N), jnp.float32)
        _gumbel_cache.append(jnp.reshape(g, (_R, _C)))
    return _gumbel_cache[0]


def _body(t_ref, g_ref, o_ref, sacc):
    p = pl.program_id(0)
    i = pl.program_id(1)

    @pl.when((p == 0) & (i == 0))
    def _():
        sacc[0] = 0.0
        sacc[1] = -jnp.inf
        sacc[2] = 0.0

    @pl.when(p == 0)
    def _():
        x = t_ref[...]
        y = x + g_ref[...]
        sacc[0] += jnp.sum(jnp.exp(x))
        m = jnp.max(y)
        rm = jax.lax.broadcasted_iota(jnp.int32, y.shape, 0) * _C \
            + jax.lax.broadcasted_iota(jnp.int32, y.shape, 1)
        first = jnp.min(jnp.where(y == m, rm, jnp.int32(2**30)))
        tv = jnp.sum(jnp.where(rm == first, x, 0.0))
        @pl.when(m > sacc[1])
        def _():
            sacc[1] = m
            sacc[2] = tv

    @pl.when(p == 1)
    def _():
        x = t_ref[...]
        lse = jnp.log(sacc[0])
        cadd = sacc[2] - 2.0 * lse
        o_ref[...] = (x + cadd) + jnp.exp(x - lse)


def kernel(t):
    out = pl.pallas_call(
        _body,
        grid=(2, _NB),
        in_specs=[
            pl.BlockSpec((_BR, _C), lambda p, i: (i, 0)),
            pl.BlockSpec((_BR, _C), lambda p, i: (i * (1 - p), 0)),
        ],
        out_specs=pl.BlockSpec((_BR, _C), lambda p, i: (i * p, 0)),
        out_shape=jax.ShapeDtypeStruct((_R, _C), jnp.float32),
        scratch_shapes=[pltpu.SMEM((3,), jnp.float32)],
    )(jnp.reshape(t, (_R, _C)), _gumbel())
    return jnp.reshape(out, (1, _N))


# speedup vs baseline: 1.3218x; 1.0509x over previous
import jax
import jax.numpy as jnp
from jax.experimental import pallas as pl
from jax.experimental.pallas import tpu as pltpu

_N = 1_000_000
_R, _C = 64, 15625
_BR = 16
_NB = _R // _BR

_gumbel_cache = []


def _gumbel():
    if not _gumbel_cache:
        g = jax.random.gumbel(jax.random.key(42), (1, _N), jnp.float32)
        _gumbel_cache.append(jnp.reshape(g, (_R, _C)))
    return _gumbel_cache[0]


def _body(t_ref, g_ref, o_ref, sacc):
    p = pl.program_id(0)
    i = pl.program_id(1)

    @pl.when((p == 0) & (i == 0))
    def _():
        sacc[0] = 0.0
        sacc[1] = -jnp.inf
        sacc[2] = 0.0

    @pl.when(p == 0)
    def _():
        x = t_ref[...]
        y = x + g_ref[...]
        sacc[0] += jnp.sum(jnp.exp(x))
        m = jnp.max(y)
        rm = jax.lax.broadcasted_iota(jnp.int32, y.shape, 0) * _C \
            + jax.lax.broadcasted_iota(jnp.int32, y.shape, 1)
        first = jnp.min(jnp.where(y == m, rm, jnp.int32(2**30)))
        tv = jnp.sum(jnp.where(rm == first, x, 0.0))
        @pl.when(m > sacc[1])
        def _():
            sacc[1] = m
            sacc[2] = tv

    @pl.when(p == 1)
    def _():
        x = t_ref[...]
        lse = jnp.log(sacc[0])
        cadd = sacc[2] - 2.0 * lse
        o_ref[...] = (x + cadd) + jnp.exp(x - lse)


def kernel(t):
    out = pl.pallas_call(
        _body,
        grid=(2, _NB),
        in_specs=[
            pl.BlockSpec((_BR, _C), lambda p, i: (i, 0)),
            pl.BlockSpec((_BR, _C), lambda p, i: (i * (1 - p), 0)),
        ],
        out_specs=pl.BlockSpec((_BR, _C), lambda p, i: (i * p, 0)),
        out_shape=jax.ShapeDtypeStruct((_R, _C), jnp.float32),
        scratch_shapes=[pltpu.SMEM((3,), jnp.float32)],
    )(jnp.reshape(t, (_R, _C)), _gumbel())
    return jnp.reshape(out, (1, _N))
